# baseline (speedup 1.0000x reference)
import jax
import jax.numpy as jnp
from jax.experimental import pallas as pl
from jax.experimental.pallas import tpu as pltpu

_NUM_CLASSES, _NUM_MAPS = 8, 4
_NC = _NUM_CLASSES * _NUM_MAPS
_NW = _NC + 2 * _NUM_CLASSES
_NP = 14
_P = 14


def _iota2(shape, d0, d1, fn):
    a = jax.lax.broadcasted_iota(jnp.int32, shape, d0)
    b = jax.lax.broadcasted_iota(jnp.int32, shape, d1)
    return fn(a, b).astype(jnp.float32)


def _wscnet_kernel(f_ref, wt_ref, cb_ref, fcb_ref, xv_ref, res_ref, y_scr):
    p = pl.program_id(1)
    bb = f_ref.shape[1]
    f2 = f_ref[...].reshape(_P * bb, 2048)
    y_scr[p] = jnp.dot(f2, wt_ref[...], preferred_element_type=jnp.float32)

    @pl.when(p == _NP - 1)
    def _finale():
        hw = _NP * _P
        y3 = y_scr[...].reshape(hw, bb, _NW)
        cb = cb_ref[...]

        gmp = jnp.max(y3, axis=0) + cb

        sel = _iota2((_NW, _NUM_CLASSES), 0, 1,
                     lambda o, k: (o // _NUM_MAPS) == k)
        xv = jnp.dot(gmp, sel, preferred_element_type=jnp.float32) * 0.25

        selt = _iota2((_NUM_CLASSES, _NW), 1, 0,
                      lambda o, k: (o // _NUM_MAPS) == k)
        coeff = jnp.dot(xv, selt,
                        preferred_element_type=jnp.float32) * (1.0 / 32.0)

        beta = jnp.sum(coeff * cb, axis=1, keepdims=True)[None]
        s3 = jnp.sum(y3 * coeff[None], axis=2, keepdims=True) + beta

        r_u1 = jnp.sum(y3, axis=0)
        r_u2 = jnp.sum(y3 * s3, axis=0)
        su1 = _iota2((_NW, _NUM_CLASSES), 0, 1, lambda o, i: o == _NC + i)
        su2 = _iota2((_NW, _NUM_CLASSES), 0, 1,
                     lambda o, i: o == _NC + _NUM_CLASSES + i)
        res = (jnp.dot(r_u1, su1, preferred_element_type=jnp.float32)
               + jnp.dot(r_u2, su2, preferred_element_type=jnp.float32)
               ) * (1.0 / hw) + fcb_ref[...]

        xv_ref[...] = xv
        res_ref[...] = res


def kernel(features, conv_w, conv_b, fc_w, fc_b):
    B, C, H, W = features.shape
    HW = H * W
    fp = features.transpose(2, 3, 0, 1).reshape(HW, B, C)
    wt = jnp.concatenate([conv_w, fc_w[:, :C], fc_w[:, C:]], axis=0).T
    cb_pad = jnp.concatenate(
        [conv_b, jnp.zeros((_NW - _NC,), jnp.float32)]).reshape(1, _NW)
    fcb2 = fc_b.reshape(1, _NUM_CLASSES)

    bb = B // 2
    out_sds = jax.ShapeDtypeStruct((B, _NUM_CLASSES), jnp.float32)
    xv, res = pl.pallas_call(
        _wscnet_kernel,
        grid=(2, _NP),
        in_specs=[
            pl.BlockSpec((_P, bb, C), lambda i, p: (p, i, 0)),
            pl.BlockSpec((C, _NW), lambda i, p: (0, 0)),
            pl.BlockSpec((1, _NW), lambda i, p: (0, 0)),
            pl.BlockSpec((1, _NUM_CLASSES), lambda i, p: (0, 0)),
        ],
        out_specs=[
            pl.BlockSpec((bb, _NUM_CLASSES), lambda i, p: (i, 0)),
            pl.BlockSpec((bb, _NUM_CLASSES), lambda i, p: (i, 0)),
        ],
        out_shape=[out_sds, out_sds],
        scratch_shapes=[pltpu.VMEM((_NP, _P * bb, _NW), jnp.float32)],
        compiler_params=pltpu.CompilerParams(
            dimension_semantics=("parallel", "arbitrary")),
    )(fp, wt, cb_pad, fcb2)
    return (xv, res)


# speedup vs baseline: 1.1668x; 1.1668x over previous
import jax
import jax.numpy as jnp
from jax.experimental import pallas as pl
from jax.experimental.pallas import tpu as pltpu

_NUM_CLASSES, _NUM_MAPS = 8, 4
_NC = _NUM_CLASSES * _NUM_MAPS
_NW = _NC + 2 * _NUM_CLASSES
_NP = 4
_P = 49


def _iota2(shape, d0, d1, fn):
    a = jax.lax.broadcasted_iota(jnp.int32, shape, d0)
    b = jax.lax.broadcasted_iota(jnp.int32, shape, d1)
    return fn(a, b).astype(jnp.float32)


def _wscnet_kernel(f_ref, wt_ref, cb_ref, fcb_ref, xv_ref, res_ref, y_scr):
    p = pl.program_id(1)
    bb = f_ref.shape[1]
    f2 = f_ref[...].reshape(_P * bb, 2048)
    y_scr[p] = jnp.dot(f2, wt_ref[...], preferred_element_type=jnp.float32)

    @pl.when(p == _NP - 1)
    def _finale():
        hw = _NP * _P
        y3 = y_scr[...].reshape(hw, bb, _NW)
        cb = cb_ref[...]

        gmp = jnp.max(y3, axis=0) + cb

        sel = _iota2((_NW, _NUM_CLASSES), 0, 1,
                     lambda o, k: (o // _NUM_MAPS) == k)
        xv = jnp.dot(gmp, sel, preferred_element_type=jnp.float32) * 0.25

        selt = _iota2((_NUM_CLASSES, _NW), 1, 0,
                      lambda o, k: (o // _NUM_MAPS) == k)
        coeff = jnp.dot(xv, selt,
                        preferred_element_type=jnp.float32) * (1.0 / 32.0)

        beta = jnp.sum(coeff * cb, axis=1, keepdims=True)[None]
        s3 = jnp.sum(y3 * coeff[None], axis=2, keepdims=True) + beta

        r_u1 = jnp.sum(y3, axis=0)
        r_u2 = jnp.sum(y3 * s3, axis=0)
        su1 = _iota2((_NW, _NUM_CLASSES), 0, 1, lambda o, i: o == _NC + i)
        su2 = _iota2((_NW, _NUM_CLASSES), 0, 1,
                     lambda o, i: o == _NC + _NUM_CLASSES + i)
        res = (jnp.dot(r_u1, su1, preferred_element_type=jnp.float32)
               + jnp.dot(r_u2, su2, preferred_element_type=jnp.float32)
               ) * (1.0 / hw) + fcb_ref[...]

        xv_ref[...] = xv
        res_ref[...] = res


def kernel(features, conv_w, conv_b, fc_w, fc_b):
    B, C, H, W = features.shape
    HW = H * W
    fp = features.transpose(2, 3, 0, 1).reshape(HW, B, C)
    wt = jnp.concatenate([conv_w, fc_w[:, :C], fc_w[:, C:]], axis=0).T
    cb_pad = jnp.concatenate(
        [conv_b, jnp.zeros((_NW - _NC,), jnp.float32)]).reshape(1, _NW)
    fcb2 = fc_b.reshape(1, _NUM_CLASSES)

    bb = B // 2
    out_sds = jax.ShapeDtypeStruct((B, _NUM_CLASSES), jnp.float32)
    xv, res = pl.pallas_call(
        _wscnet_kernel,
        grid=(2, _NP),
        in_specs=[
            pl.BlockSpec((_P, bb, C), lambda i, p: (p, i, 0)),
            pl.BlockSpec((C, _NW), lambda i, p: (0, 0)),
            pl.BlockSpec((1, _NW), lambda i, p: (0, 0)),
            pl.BlockSpec((1, _NUM_CLASSES), lambda i, p: (0, 0)),
        ],
        out_specs=[
            pl.BlockSpec((bb, _NUM_CLASSES), lambda i, p: (i, 0)),
            pl.BlockSpec((bb, _NUM_CLASSES), lambda i, p: (i, 0)),
        ],
        out_shape=[out_sds, out_sds],
        scratch_shapes=[pltpu.VMEM((_NP, _P * bb, _NW), jnp.float32)],
        compiler_params=pltpu.CompilerParams(
            dimension_semantics=("parallel", "arbitrary")),
    )(fp, wt, cb_pad, fcb2)
    return (xv, res)


# speedup vs baseline: 1.1737x; 1.0059x over previous
import jax
import jax.numpy as jnp
from jax.experimental import pallas as pl
from jax.experimental.pallas import tpu as pltpu

_NUM_CLASSES, _NUM_MAPS = 8, 4
_NC = _NUM_CLASSES * _NUM_MAPS
_NW = _NC + 2 * _NUM_CLASSES
_NP = 7
_P = 28
_HB = 16


def _iota2(shape, d0, d1, fn):
    a = jax.lax.broadcasted_iota(jnp.int32, shape, d0)
    b = jax.lax.broadcasted_iota(jnp.int32, shape, d1)
    return fn(a, b).astype(jnp.float32)


def _finale_half(y3, cb, fcb):
    hw = y3.shape[0]

    gmp = jnp.max(y3, axis=0) + cb

    sel = _iota2((_NW, _NUM_CLASSES), 0, 1,
                 lambda o, k: (o // _NUM_MAPS) == k)
    xv = jnp.dot(gmp, sel, preferred_element_type=jnp.float32) * 0.25

    selt = _iota2((_NUM_CLASSES, _NW), 1, 0,
                  lambda o, k: (o // _NUM_MAPS) == k)
    coeff = jnp.dot(xv, selt,
                    preferred_element_type=jnp.float32) * (1.0 / 32.0)

    beta = jnp.sum(coeff * cb, axis=1, keepdims=True)[None]
    s3 = jnp.sum(y3 * coeff[None], axis=2, keepdims=True) + beta

    r_u1 = jnp.sum(y3, axis=0)
    r_u2 = jnp.sum(y3 * s3, axis=0)
    su1 = _iota2((_NW, _NUM_CLASSES), 0, 1, lambda o, i: o == _NC + i)
    su2 = _iota2((_NW, _NUM_CLASSES), 0, 1,
                 lambda o, i: o == _NC + _NUM_CLASSES + i)
    res = (jnp.dot(r_u1, su1, preferred_element_type=jnp.float32)
           + jnp.dot(r_u2, su2, preferred_element_type=jnp.float32)
           ) * (1.0 / hw) + fcb
    return xv, res


def _wscnet_kernel(fa_ref, fb_ref, wt_ref, cb_ref, fcb_ref,
                   xv_ref, res_ref, y_scr):
    p = pl.program_id(1)
    wt = wt_ref[...]
    y_scr[p, 0] = jnp.dot(fa_ref[...].reshape(_P * _HB, 2048), wt,
                          preferred_element_type=jnp.float32)
    y_scr[p, 1] = jnp.dot(fb_ref[...].reshape(_P * _HB, 2048), wt,
                          preferred_element_type=jnp.float32)

    @pl.when(p == _NP - 1)
    def _finale():
        hw = _NP * _P
        cb = cb_ref[...]
        fcb = fcb_ref[...]
        for h in range(2):
            y3 = y_scr[:, h].reshape(hw, _HB, _NW)
            xv, res = _finale_half(y3, cb, fcb)
            xv_ref[h * _HB:(h + 1) * _HB] = xv
            res_ref[h * _HB:(h + 1) * _HB] = res


def kernel(features, conv_w, conv_b, fc_w, fc_b):
    B, C, H, W = features.shape
    HW = H * W
    fp = features.transpose(2, 3, 0, 1).reshape(HW, B // _HB, _HB, C)
    wt = jnp.concatenate([conv_w, fc_w[:, :C], fc_w[:, C:]], axis=0).T
    cb_pad = jnp.concatenate(
        [conv_b, jnp.zeros((_NW - _NC,), jnp.float32)]).reshape(1, _NW)
    fcb2 = fc_b.reshape(1, _NUM_CLASSES)

    bb = B // 2
    out_sds = jax.ShapeDtypeStruct((B, _NUM_CLASSES), jnp.float32)
    xv, res = pl.pallas_call(
        _wscnet_kernel,
        grid=(2, _NP),
        in_specs=[
            pl.BlockSpec((_P, 1, _HB, C), lambda i, p: (p, 2 * i, 0, 0)),
            pl.BlockSpec((_P, 1, _HB, C), lambda i, p: (p, 2 * i + 1, 0, 0)),
            pl.BlockSpec((C, _NW), lambda i, p: (0, 0)),
            pl.BlockSpec((1, _NW), lambda i, p: (0, 0)),
            pl.BlockSpec((1, _NUM_CLASSES), lambda i, p: (0, 0)),
        ],
        out_specs=[
            pl.BlockSpec((bb, _NUM_CLASSES), lambda i, p: (i, 0)),
            pl.BlockSpec((bb, _NUM_CLASSES), lambda i, p: (i, 0)),
        ],
        out_shape=[out_sds, out_sds],
        scratch_shapes=[pltpu.VMEM((_NP, 2, _P * _HB, _NW), jnp.float32)],
        compiler_params=pltpu.CompilerParams(
            dimension_semantics=("parallel", "arbitrary")),
    )(fp, fp, wt, cb_pad, fcb2)
    return (xv, res)


# speedup vs baseline: 1.1793x; 1.0047x over previous
import jax
import jax.numpy as jnp
from jax.experimental import pallas as pl
from jax.experimental.pallas import tpu as pltpu

_NUM_CLASSES, _NUM_MAPS = 8, 4
_NC = _NUM_CLASSES * _NUM_MAPS
_NW = _NC + 2 * _NUM_CLASSES
_NP = 7
_P = 28


def _iota2(shape, d0, d1, fn):
    a = jax.lax.broadcasted_iota(jnp.int32, shape, d0)
    b = jax.lax.broadcasted_iota(jnp.int32, shape, d1)
    return fn(a, b).astype(jnp.float32)


def _wscnet_kernel(f_ref, wt_ref, cb_ref, fcb_ref, xv_ref, res_ref, y_scr):
    p = pl.program_id(1)
    bb = f_ref.shape[1]
    f2 = f_ref[...].reshape(_P * bb, 2048)
    y_scr[p] = jnp.dot(f2, wt_ref[...], preferred_element_type=jnp.float32)

    @pl.when(p == _NP - 1)
    def _finale():
        hw = _NP * _P
        y3 = y_scr[...].reshape(hw, bb, _NW)
        cb = cb_ref[...]

        gmp = jnp.max(y3, axis=0) + cb

        sel = _iota2((_NW, _NUM_CLASSES), 0, 1,
                     lambda o, k: (o // _NUM_MAPS) == k)
        xv = jnp.dot(gmp, sel, preferred_element_type=jnp.float32) * 0.25

        selt = _iota2((_NUM_CLASSES, _NW), 1, 0,
                      lambda o, k: (o // _NUM_MAPS) == k)
        coeff = jnp.dot(xv, selt,
                        preferred_element_type=jnp.float32) * (1.0 / 32.0)

        beta = jnp.sum(coeff * cb, axis=1, keepdims=True)[None]
        s3 = jnp.sum(y3 * coeff[None], axis=2, keepdims=True) + beta

        r_u1 = jnp.sum(y3, axis=0)
        r_u2 = jnp.sum(y3 * s3, axis=0)
        su1 = _iota2((_NW, _NUM_CLASSES), 0, 1, lambda o, i: o == _NC + i)
        su2 = _iota2((_NW, _NUM_CLASSES), 0, 1,
                     lambda o, i: o == _NC + _NUM_CLASSES + i)
        res = (jnp.dot(r_u1, su1, preferred_element_type=jnp.float32)
               + jnp.dot(r_u2, su2, preferred_element_type=jnp.float32)
               ) * (1.0 / hw) + fcb_ref[...]

        xv_ref[...] = xv
        res_ref[...] = res


def kernel(features, conv_w, conv_b, fc_w, fc_b):
    B, C, H, W = features.shape
    HW = H * W
    fp = features.transpose(2, 3, 0, 1).reshape(HW, B, C)
    wt = jnp.concatenate([conv_w, fc_w[:, :C], fc_w[:, C:]], axis=0).T
    cb_pad = jnp.concatenate(
        [conv_b, jnp.zeros((_NW - _NC,), jnp.float32)]).reshape(1, _NW)
    fcb2 = fc_b.reshape(1, _NUM_CLASSES)

    bb = B // 2
    out_sds = jax.ShapeDtypeStruct((B, _NUM_CLASSES), jnp.float32)
    xv, res = pl.pallas_call(
        _wscnet_kernel,
        grid=(2, _NP),
        in_specs=[
            pl.BlockSpec((_P, bb, C), lambda i, p: (p, i, 0)),
            pl.BlockSpec((C, _NW), lambda i, p: (0, 0)),
            pl.BlockSpec((1, _NW), lambda i, p: (0, 0)),
            pl.BlockSpec((1, _NUM_CLASSES), lambda i, p: (0, 0)),
        ],
        out_specs=[
            pl.BlockSpec((bb, _NUM_CLASSES), lambda i, p: (i, 0)),
            pl.BlockSpec((bb, _NUM_CLASSES), lambda i, p: (i, 0)),
        ],
        out_shape=[out_sds, out_sds],
        scratch_shapes=[pltpu.VMEM((_NP, _P * bb, _NW), jnp.float32)],
        compiler_params=pltpu.CompilerParams(
            dimension_semantics=("parallel", "arbitrary")),
    )(fp, wt, cb_pad, fcb2)
    return (xv, res)


# speedup vs baseline: 1.1843x; 1.0043x over previous
import jax
import jax.numpy as jnp
from jax.experimental import pallas as pl
from jax.experimental.pallas import tpu as pltpu

_NUM_CLASSES, _NUM_MAPS = 8, 4
_NC = _NUM_CLASSES * _NUM_MAPS
_NW = _NC + 2 * _NUM_CLASSES
_NP = 7
_P = 28


def _iota2(shape, d0, d1, fn):
    a = jax.lax.broadcasted_iota(jnp.int32, shape, d0)
    b = jax.lax.broadcasted_iota(jnp.int32, shape, d1)
    return fn(a, b).astype(jnp.float32)


def _wscnet_kernel(f_ref, wt_ref, cb_ref, fcb_ref, xv_ref, res_ref, y_scr):
    p = pl.program_id(1)
    bb = f_ref.shape[1]
    f2 = f_ref[...].reshape(_P * bb, f_ref.shape[2])
    y_scr[p] = jnp.dot(f2, wt_ref[...], preferred_element_type=jnp.float32)

    @pl.when(p == _NP - 1)
    def _finale():
        hw = _NP * _P
        y3 = y_scr[...].reshape(hw, bb, _NW)
        cb = cb_ref[...]

        gmp = jnp.max(y3, axis=0) + cb

        sel = _iota2((_NW, _NUM_CLASSES), 0, 1,
                     lambda o, k: (o // _NUM_MAPS) == k)
        xv = jnp.dot(gmp, sel, preferred_element_type=jnp.float32) * 0.25

        selt = _iota2((_NUM_CLASSES, _NW), 1, 0,
                      lambda o, k: (o // _NUM_MAPS) == k)
        coeff = jnp.dot(xv, selt,
                        preferred_element_type=jnp.float32) * (1.0 / 32.0)

        beta = jnp.sum(coeff * cb, axis=1, keepdims=True)[None]
        s3 = jnp.sum(y3 * coeff[None], axis=2, keepdims=True) + beta

        r_u1 = jnp.sum(y3, axis=0)
        r_u2 = jnp.sum(y3 * s3, axis=0)
        su1 = _iota2((_NW, _NUM_CLASSES), 0, 1, lambda o, i: o == _NC + i)
        su2 = _iota2((_NW, _NUM_CLASSES), 0, 1,
                     lambda o, i: o == _NC + _NUM_CLASSES + i)
        res = (jnp.dot(r_u1, su1, preferred_element_type=jnp.float32)
               + jnp.dot(r_u2, su2, preferred_element_type=jnp.float32)
               ) * (1.0 / hw) + fcb_ref[...]

        xv_ref[...] = xv
        res_ref[...] = res


def kernel(features, conv_w, conv_b, fc_w, fc_b):
    B, C, H, W = features.shape
    HW = H * W
    fp = features.transpose(2, 3, 0, 1).reshape(HW, B, C)
    wt = jnp.concatenate([conv_w, fc_w[:, :C], fc_w[:, C:]], axis=0).T
    cb_pad = jnp.concatenate(
        [conv_b, jnp.zeros((_NW - _NC,), jnp.float32)]).reshape(1, _NW)
    fcb2 = fc_b.reshape(1, _NUM_CLASSES)

    bb = B // 2
    out_sds = jax.ShapeDtypeStruct((B, _NUM_CLASSES), jnp.float32)
    xv, res = pl.pallas_call(
        _wscnet_kernel,
        grid=(2, _NP),
        in_specs=[
            pl.BlockSpec((_P, bb, C), lambda i, p: (p, i, 0)),
            pl.BlockSpec((C, _NW), lambda i, p: (0, 0)),
            pl.BlockSpec((1, _NW), lambda i, p: (0, 0)),
            pl.BlockSpec((1, _NUM_CLASSES), lambda i, p: (0, 0)),
        ],
        out_specs=[
            pl.BlockSpec((bb, _NUM_CLASSES), lambda i, p: (i, 0)),
            pl.BlockSpec((bb, _NUM_CLASSES), lambda i, p: (i, 0)),
        ],
        out_shape=[out_sds, out_sds],
        scratch_shapes=[pltpu.VMEM((_NP, _P * bb, _NW), jnp.float32)],
        compiler_params=pltpu.CompilerParams(
            dimension_semantics=("parallel", "arbitrary")),
    )(fp, wt, cb_pad, fcb2)
    return (xv, res)
